# Initial kernel scaffold; baseline (speedup 1.0000x reference)
import jax
import jax.numpy as jnp
from jax import lax
from jax.experimental import pallas as pl
from jax.experimental.pallas import tpu as pltpu
from jax.experimental.pallas import tpu_sc as plsc

N_TOK = 16384
N_CLUSTERS = 8192
DIM = 32
TOK_TILE = 512
GATHER_WINDOW = 128


def _argmin_body(x_ref, c_ref, nh_ref, idx_ref):
    x = x_ref[...] + 1e-06
    c = c_ref[...]
    s = lax.dot_general(
        x, c, (((1,), (1,)), ((), ())),
        preferred_element_type=jnp.float32,
        precision=lax.Precision.HIGHEST,
    )
    s = s + nh_ref[...]
    maxval = jnp.max(s, axis=1, keepdims=True)
    iota = lax.broadcasted_iota(jnp.int32, s.shape, 1)
    idx = jnp.min(
        jnp.where(s == maxval, iota, jnp.int32(N_CLUSTERS)),
        axis=1, keepdims=True,
    )
    idx_ref[...] = idx


def _nearest_indices(x_flat, cluster_centers, neg_half_c2):
    return pl.pallas_call(
        _argmin_body,
        grid=(N_TOK // TOK_TILE,),
        in_specs=[
            pl.BlockSpec((TOK_TILE, DIM), lambda i: (i, 0)),
            pl.BlockSpec((N_CLUSTERS, DIM), lambda i: (0, 0)),
            pl.BlockSpec((1, N_CLUSTERS), lambda i: (0, 0)),
        ],
        out_specs=pl.BlockSpec((TOK_TILE, 1), lambda i: (i, 0)),
        out_shape=jax.ShapeDtypeStruct((N_TOK, 1), jnp.int32),
    )(x_flat, cluster_centers, neg_half_c2)


def _sc_gather(table, indices_row):
    @pl.kernel(
        out_type=jax.ShapeDtypeStruct((N_TOK, DIM), table.dtype),
        mesh=plsc.VectorSubcoreMesh(
            core_axis_name="core", subcore_axis_name="subcore"
        ),
    )
    def gather_kernel(tab_hbm, i_hbm, o_hbm):
        def body(i_vmem, o_vmem):
            pltpu.sync_copy(tab_hbm.at[i_vmem.at[0]], o_vmem)

        pltpu.emit_pipeline(
            body,
            grid=(N_TOK // GATHER_WINDOW,),
            in_specs=[
                pl.BlockSpec((1, GATHER_WINDOW), index_map=lambda i: (0, i))
            ],
            out_specs=[
                pl.BlockSpec((GATHER_WINDOW, DIM), index_map=lambda i: (i, 0))
            ],
            core_axis_name=("core", "subcore"),
            dimension_semantics=(pltpu.PARALLEL,),
        )(i_hbm, o_hbm)

    return gather_kernel(table, indices_row)


def kernel(x, cluster_centers):
    b, t, d = x.shape
    x_flat = x.reshape(b * t, d)
    neg_half_c2 = (-0.5 * jnp.sum(cluster_centers * cluster_centers,
                                  axis=-1))[None, :]
    idx = _nearest_indices(x_flat, cluster_centers, neg_half_c2)
    out = _sc_gather(cluster_centers, idx.reshape(1, N_TOK))
    return out.reshape(b, t, d)


# speedup vs baseline: 1.8616x; 1.8616x over previous
import jax
import jax.numpy as jnp
from jax import lax
from jax.experimental import pallas as pl
from jax.experimental.pallas import tpu as pltpu
from jax.experimental.pallas import tpu_sc as plsc

N_TOK = 16384
N_CLUSTERS = 8192
DIM = 32
TOK_TILE = 512
GATHER_WINDOW = 128


def _argmin_body(x_ref, c_ref, c2_ref, idx_ref):
    xs = x_ref[...] + 1e-06
    c = c_ref[...]
    x2 = jnp.sum(xs * xs, axis=1, keepdims=True)
    cross = lax.dot_general(
        xs, c, (((1,), (1,)), ((), ())),
        preferred_element_type=jnp.float32,
        precision=lax.Precision.DEFAULT,
    )
    d2 = (x2 + c2_ref[...]) - 2.0 * cross
    minval = jnp.min(d2, axis=1, keepdims=True)
    iota = lax.broadcasted_iota(jnp.int32, d2.shape, 1)
    idx = jnp.min(
        jnp.where(d2 == minval, iota, jnp.int32(N_CLUSTERS)),
        axis=1, keepdims=True,
    )
    idx_ref[...] = idx


def _nearest_indices(x_flat, cluster_centers, c2_row):
    return pl.pallas_call(
        _argmin_body,
        grid=(N_TOK // TOK_TILE,),
        in_specs=[
            pl.BlockSpec((TOK_TILE, DIM), lambda i: (i, 0)),
            pl.BlockSpec((N_CLUSTERS, DIM), lambda i: (0, 0)),
            pl.BlockSpec((1, N_CLUSTERS), lambda i: (0, 0)),
        ],
        out_specs=pl.BlockSpec((TOK_TILE, 1), lambda i: (i, 0)),
        out_shape=jax.ShapeDtypeStruct((N_TOK, 1), jnp.int32),
    )(x_flat, cluster_centers, c2_row)


def _sc_gather(table, indices_row):
    @pl.kernel(
        out_type=jax.ShapeDtypeStruct((N_TOK, 128), table.dtype),
        mesh=plsc.VectorSubcoreMesh(
            core_axis_name="core", subcore_axis_name="subcore"
        ),
    )
    def gather_kernel(tab_hbm, i_hbm, o_hbm):
        def body(i_vmem, o_vmem):
            pltpu.sync_copy(tab_hbm.at[i_vmem.at[0]], o_vmem)

        pltpu.emit_pipeline(
            body,
            grid=(N_TOK // GATHER_WINDOW,),
            in_specs=[
                pl.BlockSpec((1, GATHER_WINDOW), index_map=lambda i: (0, i))
            ],
            out_specs=[
                pl.BlockSpec((GATHER_WINDOW, 128), index_map=lambda i: (i, 0))
            ],
            core_axis_name=("core", "subcore"),
            dimension_semantics=(pltpu.PARALLEL,),
        )(i_hbm, o_hbm)

    return gather_kernel(table, indices_row)


def kernel(x, cluster_centers):
    b, t, d = x.shape
    x_flat = x.reshape(b * t, d)
    c2_row = jnp.sum(cluster_centers * cluster_centers, axis=-1)[None, :]
    idx = _nearest_indices(x_flat, cluster_centers, c2_row)
    table_pad = jnp.pad(cluster_centers, ((0, 0), (0, 128 - DIM)))
    out = _sc_gather(table_pad, idx.reshape(1, N_TOK))
    return out[:, :DIM].reshape(b, t, d)


# speedup vs baseline: 2.1697x; 1.1655x over previous
import jax
import jax.numpy as jnp
from jax import lax
from jax.experimental import pallas as pl
from jax.experimental.pallas import tpu as pltpu
from jax.experimental.pallas import tpu_sc as plsc

N_TOK = 16384
N_CLUSTERS = 8192
DIM = 32
TOK_TILE = 512
GATHER_WINDOW = 128


ROW_TILE = 64
LANES = 128
N_CHUNKS = N_CLUSTERS // LANES


def _argmin_body(x_ref, c_ref, c2_ref, idx_ref):
    xs = x_ref[...] + 1e-06
    c = c_ref[...]
    x2 = jnp.sum(xs * xs, axis=1, keepdims=True)
    cross2 = lax.dot_general(
        xs, c + c, (((1,), (1,)), ((), ())),
        preferred_element_type=jnp.float32,
        precision=lax.Precision.DEFAULT,
    )
    c2r = c2_ref[...]
    lane_iota = lax.broadcasted_iota(jnp.int32, (ROW_TILE, LANES), 1)
    for r in range(TOK_TILE // ROW_TILE):
        rows = slice(r * ROW_TILE, (r + 1) * ROW_TILE)
        x2r = x2[rows, :]
        bestv = (x2r + c2r[:, 0:LANES]) - cross2[rows, 0:LANES]
        bestj = jnp.zeros((ROW_TILE, LANES), jnp.int32)
        for j in range(1, N_CHUNKS):
            cols = slice(j * LANES, (j + 1) * LANES)
            v = (x2r + c2r[:, cols]) - cross2[rows, cols]
            lt = v < bestv
            bestv = jnp.where(lt, v, bestv)
            bestj = jnp.where(lt, jnp.int32(j), bestj)
        rowmin = jnp.min(bestv, axis=1, keepdims=True)
        k = bestj * LANES + lane_iota
        cand = jnp.where(bestv == rowmin, k, jnp.int32(N_CLUSTERS * 2))
        idx_ref[rows, :] = jnp.min(cand, axis=1, keepdims=True)


def _nearest_indices(x_flat, cluster_centers, c2_row):
    return pl.pallas_call(
        _argmin_body,
        grid=(N_TOK // TOK_TILE,),
        in_specs=[
            pl.BlockSpec((TOK_TILE, DIM), lambda i: (i, 0)),
            pl.BlockSpec((N_CLUSTERS, DIM), lambda i: (0, 0)),
            pl.BlockSpec((1, N_CLUSTERS), lambda i: (0, 0)),
        ],
        out_specs=pl.BlockSpec((TOK_TILE, 1), lambda i: (i, 0)),
        out_shape=jax.ShapeDtypeStruct((N_TOK, 1), jnp.int32),
    )(x_flat, cluster_centers, c2_row)


def _sc_gather(table, indices_row):
    @pl.kernel(
        out_type=jax.ShapeDtypeStruct((N_TOK, 128), table.dtype),
        mesh=plsc.VectorSubcoreMesh(
            core_axis_name="core", subcore_axis_name="subcore"
        ),
    )
    def gather_kernel(tab_hbm, i_hbm, o_hbm):
        def body(i_vmem, o_vmem):
            pltpu.sync_copy(tab_hbm.at[i_vmem.at[0]], o_vmem)

        pltpu.emit_pipeline(
            body,
            grid=(N_TOK // GATHER_WINDOW,),
            in_specs=[
                pl.BlockSpec((1, GATHER_WINDOW), index_map=lambda i: (0, i))
            ],
            out_specs=[
                pl.BlockSpec((GATHER_WINDOW, 128), index_map=lambda i: (i, 0))
            ],
            core_axis_name=("core", "subcore"),
            dimension_semantics=(pltpu.PARALLEL,),
        )(i_hbm, o_hbm)

    return gather_kernel(table, indices_row)


def kernel(x, cluster_centers):
    b, t, d = x.shape
    x_flat = x.reshape(b * t, d)
    c2_row = jnp.sum(cluster_centers * cluster_centers, axis=-1)[None, :]
    idx = _nearest_indices(x_flat, cluster_centers, c2_row)
    table_pad = jnp.pad(cluster_centers, ((0, 0), (0, 128 - DIM)))
    out = _sc_gather(table_pad, idx.reshape(1, N_TOK))
    return out[:, :DIM].reshape(b, t, d)


# speedup vs baseline: 2.6198x; 1.2074x over previous
import jax
import jax.numpy as jnp
from jax import lax
from jax.experimental import pallas as pl
from jax.experimental.pallas import tpu as pltpu
from jax.experimental.pallas import tpu_sc as plsc

N_TOK = 16384
N_CLUSTERS = 8192
DIM = 32
TOK_TILE = 512
GATHER_WINDOW = 128


ROW_TILE = 64
LANES = 128
N_CHUNKS = N_CLUSTERS // LANES


def _argmin_body(x_ref, b_ref, idx_ref):
    xs = x_ref[...] + 1e-06
    x2 = jnp.sum(xs * xs, axis=1, keepdims=True)
    a = jnp.concatenate(
        [xs, x2, jnp.ones((TOK_TILE, 3), jnp.float32)], axis=1)
    t = lax.dot_general(
        a, b_ref[...], (((1,), (1,)), ((), ())),
        preferred_element_type=jnp.float32,
        precision=lax.Precision.DEFAULT,
    )
    lane_iota = lax.broadcasted_iota(jnp.int32, (ROW_TILE, LANES), 1)
    for r in range(TOK_TILE // ROW_TILE):
        rows = slice(r * ROW_TILE, (r + 1) * ROW_TILE)
        bestv = t[rows, 0:LANES]
        bestj = jnp.zeros((ROW_TILE, LANES), jnp.int32)
        for j in range(1, N_CHUNKS):
            v = t[rows, j * LANES:(j + 1) * LANES]
            lt = v < bestv
            bestv = jnp.where(lt, v, bestv)
            bestj = jnp.where(lt, jnp.int32(j), bestj)
        rowmin = jnp.min(bestv, axis=1, keepdims=True)
        k = bestj * LANES + lane_iota
        cand = jnp.where(bestv == rowmin, k, jnp.int32(N_CLUSTERS * 2))
        idx_ref[rows, :] = jnp.min(cand, axis=1, keepdims=True)


def _nearest_indices(x_flat, b_mat):
    return pl.pallas_call(
        _argmin_body,
        grid=(N_TOK // TOK_TILE,),
        in_specs=[
            pl.BlockSpec((TOK_TILE, DIM), lambda i: (i, 0)),
            pl.BlockSpec((N_CLUSTERS, DIM + 4), lambda i: (0, 0)),
        ],
        out_specs=pl.BlockSpec((TOK_TILE, 1), lambda i: (i, 0)),
        out_shape=jax.ShapeDtypeStruct((N_TOK, 1), jnp.int32),
    )(x_flat, b_mat)


def _sc_gather(table, indices_row):
    @pl.kernel(
        out_type=jax.ShapeDtypeStruct((N_TOK, 128), table.dtype),
        mesh=plsc.VectorSubcoreMesh(
            core_axis_name="core", subcore_axis_name="subcore"
        ),
    )
    def gather_kernel(tab_hbm, i_hbm, o_hbm):
        def body(i_vmem, o_vmem):
            pltpu.sync_copy(tab_hbm.at[i_vmem.at[0]], o_vmem)

        pltpu.emit_pipeline(
            body,
            grid=(N_TOK // GATHER_WINDOW,),
            in_specs=[
                pl.BlockSpec((1, GATHER_WINDOW), index_map=lambda i: (0, i))
            ],
            out_specs=[
                pl.BlockSpec((GATHER_WINDOW, 128), index_map=lambda i: (i, 0))
            ],
            core_axis_name=("core", "subcore"),
            dimension_semantics=(pltpu.PARALLEL,),
        )(i_hbm, o_hbm)

    return gather_kernel(table, indices_row)


def kernel(x, cluster_centers):
    b, t, d = x.shape
    x_flat = x.reshape(b * t, d)
    c2 = jnp.sum(cluster_centers * cluster_centers, axis=-1, keepdims=True)
    hi = c2.astype(jnp.bfloat16).astype(jnp.float32)
    r1 = c2 - hi
    mid = r1.astype(jnp.bfloat16).astype(jnp.float32)
    lo = (r1 - mid).astype(jnp.bfloat16).astype(jnp.float32)
    b_mat = jnp.concatenate(
        [-2.0 * cluster_centers,
         jnp.ones((N_CLUSTERS, 1), jnp.float32), hi, mid, lo], axis=1)
    idx = _nearest_indices(x_flat, b_mat)
    table_pad = jnp.pad(cluster_centers, ((0, 0), (0, 128 - DIM)))
    out = _sc_gather(table_pad, idx.reshape(1, N_TOK))
    return out[:, :DIM].reshape(b, t, d)
